# Initial kernel scaffold; baseline (speedup 1.0000x reference)
import jax
import jax.numpy as jnp
from jax.experimental import pallas as pl


def kernel(feat, edge_index, edge_weight, W1, W2, b1, b2, prelu_a, subgraph_size):
    raise NotImplementedError("write your pallas kernel here")


# speedup vs baseline: 5.0123x; 5.0123x over previous
import functools

import jax
import jax.numpy as jnp
from jax import lax
from jax.experimental import pallas as pl
from jax.experimental.pallas import tpu as pltpu
from jax.experimental.pallas import tpu_sc as plsc

N = 10000
E = 320000
D_IN = 128
D_OUT = 256
S = 10
B = N // S

NC = 2
NS = 16
NW = NC * NS

CH = 125
ROWS = E // CH
RPW = ROWS // NW
RPT = ROWS // NS
CHV = 128
NPAD = 10240
HPT = NPAD // NS
ACC_RPT = N // NS
WB_R = 125

def _rsqrt16(x):
    i = plsc.bitcast(x, jnp.int32)
    i = jnp.int32(0x5F3759DF) - lax.shift_right_logical(i, 1)
    y = plsc.bitcast(i, jnp.float32)
    for _ in range(3):
        y = y * (1.5 - 0.5 * x * y * y)
    return y


def _sc_conv_body(edges_hbm, w_hbm, feat_hbm,
                  part_hbm, deg_hbm,
                  sA, w_v, ones_v, nb, rows_v, wp_v, rv_c,
                  hist_sh, dhist_sh, acc_sh, gsem):
    c = lax.axis_index("c")
    s = lax.axis_index("s")
    wid = s * NC + c

    zeros16 = jnp.zeros((16,), jnp.float32)
    ones16 = jnp.ones((16,), jnp.float32)
    for k in range(CHV // 16):
        ones_v[pl.ds(k * 16, 16)] = ones16

    def _z_nb(i, _):
        nb[pl.ds(i * 16, 16)] = zeros16
        return ()
    lax.fori_loop(0, HPT // 16, _z_nb, ())

    def _z_rows(i, _):
        for q in range(D_IN // 16):
            rows_v[i, pl.ds(q * 16, 16)] = zeros16
        return ()
    lax.fori_loop(0, WB_R, _z_rows, ())

    pltpu.sync_copy(nb, hist_sh.at[pl.ds(s * HPT, HPT)])
    pltpu.sync_copy(nb, dhist_sh.at[pl.ds(s * HPT, HPT)])
    for t in range(ACC_RPT // WB_R):
        pltpu.sync_copy(rows_v, acc_sh.at[pl.ds(s * ACC_RPT + t * WB_R, WB_R)])

    pltpu.sync_copy(edges_hbm.at[0, pl.ds(s * RPT, RPT)], sA)
    plsc.subcore_barrier()

    def _hist_src(j, _):
        pltpu.sync_copy(ones_v.at[pl.ds(0, CH)], hist_sh.at[sA.at[j]], add=True)
        return ()
    lax.fori_loop(0, RPT, _hist_src, ())

    pltpu.sync_copy(edges_hbm.at[1, pl.ds(c * (ROWS // NC) + s * RPW, RPW)],
                    sA.at[pl.ds(0, RPW)])

    def _hist_dst(j, _):
        pltpu.sync_copy(ones_v.at[pl.ds(0, CH)], dhist_sh.at[sA.at[j]], add=True)
        return ()
    lax.fori_loop(0, RPW, _hist_dst, ())
    plsc.subcore_barrier()

    pltpu.sync_copy(dhist_sh.at[pl.ds(s * HPT, HPT)],
                    deg_hbm.at[c, pl.ds(s * HPT, HPT)])
    pltpu.sync_copy(hist_sh.at[pl.ds(s * HPT, HPT)], nb)

    def _newton(i, _):
        x = nb[pl.ds(i * 16, 16)]
        x = jnp.maximum(x, 1.0)
        nb[pl.ds(i * 16, 16)] = _rsqrt16(x)
        return ()
    lax.fori_loop(0, HPT // 16, _newton, ())
    pltpu.sync_copy(nb, hist_sh.at[pl.ds(s * HPT, HPT)])
    plsc.subcore_barrier()

    pltpu.sync_copy(edges_hbm.at[0, pl.ds(wid * RPW, RPW)], sA.at[pl.ds(0, RPW)])
    pltpu.sync_copy(edges_hbm.at[1, pl.ds(wid * RPW, RPW)], sA.at[pl.ds(RPW, RPW)])
    pltpu.sync_copy(w_hbm.at[pl.ds(wid * RPW, RPW)], w_v)

    def _edge_chunk(j, _):
        pltpu.async_copy(feat_hbm.at[sA.at[j]], rows_v, gsem).wait()
        pltpu.sync_copy(hist_sh.at[sA.at[j]], rv_c.at[pl.ds(0, CH)])
        for off in (0, 16, 32, 48, 64, 80, 96, CH - 16):
            sv = sA[j, pl.ds(off, 16)]
            g = rv_c[pl.ds(off, 16)]
            w16 = w_v[j, pl.ds(off, 16)]
            keep = lax.rem(sv, jnp.int32(S)) != 0
            wp_v[pl.ds(off, 16)] = jnp.where(keep, w16 * g, 0.0)

        def _scale(e, _):
            g16 = plsc.load_gather(wp_v, [jnp.full((16,), e, jnp.int32)])
            for q in range(D_IN // 16):
                rows_v[e, pl.ds(q * 16, 16)] = rows_v[e, pl.ds(q * 16, 16)] * g16
            return ()
        lax.fori_loop(0, CH, _scale, ())

        pltpu.sync_copy(rows_v, acc_sh.at[sA.at[RPW + j]], add=True)
        return ()
    lax.fori_loop(0, RPW, _edge_chunk, ())
    plsc.subcore_barrier()

    for t in range(ACC_RPT // WB_R):
        blk = pl.ds(s * ACC_RPT + t * WB_R, WB_R)
        pltpu.sync_copy(acc_sh.at[blk], rows_v)
        pltpu.sync_copy(rows_v, part_hbm.at[c, blk])


@functools.partial(jax.jit, static_argnames=())
def _sc_conv(edges2d, w2d, feat):
    mesh = plsc.VectorSubcoreMesh(core_axis_name="c", subcore_axis_name="s",
                                  num_cores=NC, num_subcores=NS)
    fn = pl.kernel(
        _sc_conv_body,
        out_type=(
            jax.ShapeDtypeStruct((NC, N, D_IN), jnp.float32),
            jax.ShapeDtypeStruct((NC, NPAD), jnp.float32),
        ),
        mesh=mesh,
        scratch_types=[
            pltpu.VMEM((RPT, CH), jnp.int32),
            pltpu.VMEM((RPW, CH), jnp.float32),
            pltpu.VMEM((CHV,), jnp.float32),
            pltpu.VMEM((HPT,), jnp.float32),
            pltpu.VMEM((CH, D_IN), jnp.float32),
            pltpu.VMEM((CHV,), jnp.float32),
            pltpu.VMEM((CHV,), jnp.float32),
            pltpu.VMEM_SHARED((NPAD,), jnp.float32),
            pltpu.VMEM_SHARED((NPAD,), jnp.float32),
            pltpu.VMEM_SHARED((N, D_IN), jnp.float32),
            pltpu.SemaphoreType.DMA,
        ],
        compiler_params=pltpu.CompilerParams(use_tc_tiling_on_sc=False,
                                             needs_layout_passes=False),
    )
    return fn(edges2d, w2d, feat)


G = 200
GRID = B // G


def _tc_readout_body(p0_ref, p1_ref, d0_ref, d1_ref, fa_ref,
                     w1_ref, w2_ref, b1_ref, b2_ref, a_ref,
                     pool_ref, rec_ref, ao1_ref, ao2_ref):
    a = a_ref[0, 0]
    w1 = w1_ref[...]
    w2 = w2_ref[...]
    b1 = b1_ref[0:1, :]
    b2 = b2_ref[0:1, :]

    def prelu(x):
        return jnp.where(x >= 0, x, a * x)

    def l2n(x):
        n = jnp.sqrt(jnp.sum(x * x, axis=1, keepdims=True))
        return x / jnp.maximum(n, 1e-12)

    d = d0_ref[:, :S] + d1_ref[:, :S]
    rin = lax.rsqrt(jnp.maximum(d, 1.0))

    acc = jnp.zeros((G, D_OUT), jnp.float32)
    rec = None
    for si in range(S):
        h = (p0_ref[:, si, :] + p1_ref[:, si, :]) * rin[:, si:si + 1]
        if si == 0:
            rec = prelu(jnp.dot(h, w2, preferred_element_type=jnp.float32) + b2)
        else:
            acc = acc + prelu(jnp.dot(h, w1, preferred_element_type=jnp.float32) + b1)
    pool = acc * jnp.float32(1.0 / (S - 1))

    fa = fa_ref[...]
    ao1 = prelu(jnp.dot(fa, w1, preferred_element_type=jnp.float32) + b1)
    ao2 = prelu(jnp.dot(fa, w2, preferred_element_type=jnp.float32) + b2)

    pool_ref[...] = l2n(pool)
    rec_ref[...] = l2n(rec)
    ao1_ref[...] = l2n(ao1)
    ao2_ref[...] = l2n(ao2)


def _tc_readout(p0, p1, d0p, d1p, fa, W1, W2, b1p, b2p, ap):
    out_sds = jax.ShapeDtypeStruct((B, D_OUT), jnp.float32)
    return pl.pallas_call(
        _tc_readout_body,
        grid=(GRID,),
        in_specs=[
            pl.BlockSpec((G, S, D_IN), lambda i: (i, 0, 0)),
            pl.BlockSpec((G, S, D_IN), lambda i: (i, 0, 0)),
            pl.BlockSpec((G, 128), lambda i: (i, 0)),
            pl.BlockSpec((G, 128), lambda i: (i, 0)),
            pl.BlockSpec((G, D_IN), lambda i: (i, 0)),
            pl.BlockSpec((D_IN, D_OUT), lambda i: (0, 0)),
            pl.BlockSpec((D_IN, D_OUT), lambda i: (0, 0)),
            pl.BlockSpec((8, D_OUT), lambda i: (0, 0)),
            pl.BlockSpec((8, D_OUT), lambda i: (0, 0)),
            pl.BlockSpec((8, 128), lambda i: (0, 0)),
        ],
        out_specs=[pl.BlockSpec((G, D_OUT), lambda i: (i, 0))] * 4,
        out_shape=[out_sds] * 4,
    )(p0, p1, d0p, d1p, fa, W1, W2, b1p, b2p, ap)


def kernel(feat, edge_index, edge_weight, W1, W2, b1, b2, prelu_a, subgraph_size):
    del subgraph_size

    edges2d = edge_index.reshape(2, ROWS, CH)
    w2d = edge_weight.reshape(ROWS, CH)

    part, deg = _sc_conv(edges2d, w2d, feat)

    p0 = part[0].reshape(B, S, D_IN)
    p1 = part[1].reshape(B, S, D_IN)
    d0p = jnp.pad(deg[0, :N].reshape(B, S), ((0, 0), (0, 128 - S)))
    d1p = jnp.pad(deg[1, :N].reshape(B, S), ((0, 0), (0, 128 - S)))
    fa = feat.reshape(B, S, D_IN)[:, 0, :]
    b1p = jnp.broadcast_to(b1.reshape(1, D_OUT), (8, D_OUT))
    b2p = jnp.broadcast_to(b2.reshape(1, D_OUT), (8, D_OUT))
    ap = jnp.broadcast_to(prelu_a.reshape(1, 1), (8, 128))

    pool, rec, ao1, ao2 = _tc_readout(p0, p1, d0p, d1p, fa, W1, W2, b1p, b2p, ap)
    return pool, rec, ao1, ao2


# speedup vs baseline: 6.3479x; 1.2665x over previous
import functools

import jax
import jax.numpy as jnp
from jax import lax
from jax.experimental import pallas as pl
from jax.experimental.pallas import tpu as pltpu
from jax.experimental.pallas import tpu_sc as plsc

N = 10000
E = 320000
D_IN = 128
D_OUT = 256
S = 10
B = N // S

NC = 2
NS = 16
NW = NC * NS

CH = 125
ROWS = E // CH
RPW = ROWS // NW
RPT = ROWS // NS
CHV = 128
NPAD = 10240
HPT = NPAD // NS
ACC_RPT = N // NS
WB_R = 125

def _rsqrt16(x):
    i = plsc.bitcast(x, jnp.int32)
    i = jnp.int32(0x5F3759DF) - lax.shift_right_logical(i, 1)
    y = plsc.bitcast(i, jnp.float32)
    for _ in range(3):
        y = y * (1.5 - 0.5 * x * y * y)
    return y


GP = 16

_OFFS = (0, 16, 32, 48, 64, 80, 96, CH - 16)


def _sc_conv_body(edges_hbm, w_hbm, feat_hbm,
                  part_hbm, deg_hbm,
                  ig_v, w_v, ones_v, nb, rows0, rows1, wp_v, rv_c,
                  hist_sh, dhist_sh, acc_sh, sem0, sem1):
    c = lax.axis_index("c")
    s = lax.axis_index("s")
    wid = s * NC + c

    zeros16 = jnp.zeros((16,), jnp.float32)
    ones16 = jnp.ones((16,), jnp.float32)
    for k in range(CHV // 16):
        ones_v[pl.ds(k * 16, 16)] = ones16

    def _z_nb(i, _):
        nb[pl.ds(i * 16, 16)] = zeros16
        return ()
    lax.fori_loop(0, HPT // 16, _z_nb, ())

    def _z_rows(i, _):
        for q in range(D_IN // 16):
            rows0[i, pl.ds(q * 16, 16)] = zeros16
        return ()
    lax.fori_loop(0, WB_R, _z_rows, ())

    pltpu.sync_copy(nb, hist_sh.at[pl.ds(s * HPT, HPT)])
    pltpu.sync_copy(nb, dhist_sh.at[pl.ds(s * HPT, HPT)])
    for t in range(ACC_RPT // WB_R):
        pltpu.sync_copy(rows0, acc_sh.at[pl.ds(s * ACC_RPT + t * WB_R, WB_R)])
    plsc.subcore_barrier()

    def _hist(plane, base, n_groups, hist):
        def _outer(g, _):
            pltpu.sync_copy(edges_hbm.at[plane, pl.ds(base + g * GP, GP)],
                            ig_v.at[pl.ds(0, GP)])

            def _inner(j, _):
                pltpu.sync_copy(ones_v.at[pl.ds(0, CH)],
                                hist.at[ig_v.at[j]], add=True)
                return ()
            lax.fori_loop(0, GP, _inner, ())
            return ()
        lax.fori_loop(0, n_groups, _outer, ())

    _hist(0, s * RPT, RPT // GP, hist_sh)
    _hist(1, c * (ROWS // NC) + s * RPW, RPW // GP, dhist_sh)
    plsc.subcore_barrier()

    pltpu.sync_copy(dhist_sh.at[pl.ds(s * HPT, HPT)],
                    deg_hbm.at[c, pl.ds(s * HPT, HPT)])
    pltpu.sync_copy(hist_sh.at[pl.ds(s * HPT, HPT)], nb)

    def _newton(i, _):
        x = nb[pl.ds(i * 16, 16)]
        x = jnp.maximum(x, 1.0)
        nb[pl.ds(i * 16, 16)] = _rsqrt16(x)
        return ()
    lax.fori_loop(0, HPT // 16, _newton, ())
    pltpu.sync_copy(nb, hist_sh.at[pl.ds(s * HPT, HPT)])
    plsc.subcore_barrier()

    pltpu.sync_copy(w_hbm.at[pl.ds(wid * RPW, RPW)], w_v)
    bufs = (rows0, rows1)
    sems = (sem0, sem1)

    def _group(g, _):
        base = wid * RPW + g * GP
        pltpu.sync_copy(edges_hbm.at[0, pl.ds(base, GP)], ig_v.at[pl.ds(0, GP)])
        pltpu.sync_copy(edges_hbm.at[1, pl.ds(base, GP)], ig_v.at[pl.ds(GP, GP)])

        descs = {0: pltpu.async_copy(feat_hbm.at[ig_v.at[0]], bufs[0], sems[0])}
        for j2 in range(GP):
            cur = bufs[j2 % 2]
            if j2 + 1 < GP:
                descs[j2 + 1] = pltpu.async_copy(
                    feat_hbm.at[ig_v.at[j2 + 1]], bufs[(j2 + 1) % 2],
                    sems[(j2 + 1) % 2])

            pltpu.sync_copy(hist_sh.at[ig_v.at[j2]], rv_c.at[pl.ds(0, CH)])
            j = g * GP + j2
            for off in _OFFS:
                sv = ig_v[j2, pl.ds(off, 16)]
                gg = rv_c[pl.ds(off, 16)]
                w16 = w_v[j, pl.ds(off, 16)]
                keep = lax.rem(sv, jnp.int32(S)) != 0
                wp_v[pl.ds(off, 16)] = jnp.where(keep, w16 * gg, 0.0)

            descs.pop(j2).wait()

            def _scale(e, _, cur=cur):
                g16 = plsc.load_gather(wp_v, [jnp.full((16,), e, jnp.int32)])
                for q in range(D_IN // 16):
                    cur[e, pl.ds(q * 16, 16)] = cur[e, pl.ds(q * 16, 16)] * g16
                return ()
            lax.fori_loop(0, CH, _scale, ())

            pltpu.sync_copy(cur, acc_sh.at[ig_v.at[GP + j2]], add=True)
        return ()
    lax.fori_loop(0, RPW // GP, _group, ())
    plsc.subcore_barrier()

    for t in range(ACC_RPT // WB_R):
        blk = pl.ds(s * ACC_RPT + t * WB_R, WB_R)
        pltpu.sync_copy(acc_sh.at[blk], rows0)
        pltpu.sync_copy(rows0, part_hbm.at[c, blk])


@functools.partial(jax.jit, static_argnames=())
def _sc_conv(edges2d, w2d, feat):
    mesh = plsc.VectorSubcoreMesh(core_axis_name="c", subcore_axis_name="s",
                                  num_cores=NC, num_subcores=NS)
    fn = pl.kernel(
        _sc_conv_body,
        out_type=(
            jax.ShapeDtypeStruct((NC, N, D_IN), jnp.float32),
            jax.ShapeDtypeStruct((NC, NPAD), jnp.float32),
        ),
        mesh=mesh,
        scratch_types=[
            pltpu.VMEM((2 * GP, CH), jnp.int32),
            pltpu.VMEM((RPW, CH), jnp.float32),
            pltpu.VMEM((CHV,), jnp.float32),
            pltpu.VMEM((HPT,), jnp.float32),
            pltpu.VMEM((CH, D_IN), jnp.float32),
            pltpu.VMEM((CH, D_IN), jnp.float32),
            pltpu.VMEM((CHV,), jnp.float32),
            pltpu.VMEM((CHV,), jnp.float32),
            pltpu.VMEM_SHARED((NPAD,), jnp.float32),
            pltpu.VMEM_SHARED((NPAD,), jnp.float32),
            pltpu.VMEM_SHARED((N, D_IN), jnp.float32),
            pltpu.SemaphoreType.DMA,
            pltpu.SemaphoreType.DMA,
        ],
        compiler_params=pltpu.CompilerParams(use_tc_tiling_on_sc=False,
                                             needs_layout_passes=False),
    )
    return fn(edges2d, w2d, feat)


G = 200
GRID = B // G


def _tc_readout_body(p0_ref, p1_ref, d0_ref, d1_ref, fa_ref,
                     w1_ref, w2_ref, b1_ref, b2_ref, a_ref,
                     pool_ref, rec_ref, ao1_ref, ao2_ref):
    a = a_ref[0, 0]
    w1 = w1_ref[...]
    w2 = w2_ref[...]
    b1 = b1_ref[0:1, :]
    b2 = b2_ref[0:1, :]

    def prelu(x):
        return jnp.where(x >= 0, x, a * x)

    def l2n(x):
        n = jnp.sqrt(jnp.sum(x * x, axis=1, keepdims=True))
        return x / jnp.maximum(n, 1e-12)

    d = d0_ref[:, :S] + d1_ref[:, :S]
    rin = lax.rsqrt(jnp.maximum(d, 1.0))

    acc = jnp.zeros((G, D_OUT), jnp.float32)
    rec = None
    for si in range(S):
        h = (p0_ref[:, si, :] + p1_ref[:, si, :]) * rin[:, si:si + 1]
        if si == 0:
            rec = prelu(jnp.dot(h, w2, preferred_element_type=jnp.float32) + b2)
        else:
            acc = acc + prelu(jnp.dot(h, w1, preferred_element_type=jnp.float32) + b1)
    pool = acc * jnp.float32(1.0 / (S - 1))

    fa = fa_ref[...]
    ao1 = prelu(jnp.dot(fa, w1, preferred_element_type=jnp.float32) + b1)
    ao2 = prelu(jnp.dot(fa, w2, preferred_element_type=jnp.float32) + b2)

    pool_ref[...] = l2n(pool)
    rec_ref[...] = l2n(rec)
    ao1_ref[...] = l2n(ao1)
    ao2_ref[...] = l2n(ao2)


def _tc_readout(p0, p1, d0p, d1p, fa, W1, W2, b1p, b2p, ap):
    out_sds = jax.ShapeDtypeStruct((B, D_OUT), jnp.float32)
    return pl.pallas_call(
        _tc_readout_body,
        grid=(GRID,),
        in_specs=[
            pl.BlockSpec((G, S, D_IN), lambda i: (i, 0, 0)),
            pl.BlockSpec((G, S, D_IN), lambda i: (i, 0, 0)),
            pl.BlockSpec((G, 128), lambda i: (i, 0)),
            pl.BlockSpec((G, 128), lambda i: (i, 0)),
            pl.BlockSpec((G, D_IN), lambda i: (i, 0)),
            pl.BlockSpec((D_IN, D_OUT), lambda i: (0, 0)),
            pl.BlockSpec((D_IN, D_OUT), lambda i: (0, 0)),
            pl.BlockSpec((8, D_OUT), lambda i: (0, 0)),
            pl.BlockSpec((8, D_OUT), lambda i: (0, 0)),
            pl.BlockSpec((8, 128), lambda i: (0, 0)),
        ],
        out_specs=[pl.BlockSpec((G, D_OUT), lambda i: (i, 0))] * 4,
        out_shape=[out_sds] * 4,
    )(p0, p1, d0p, d1p, fa, W1, W2, b1p, b2p, ap)


def kernel(feat, edge_index, edge_weight, W1, W2, b1, b2, prelu_a, subgraph_size):
    del subgraph_size

    edges2d = edge_index.reshape(2, ROWS, CH)
    w2d = edge_weight.reshape(ROWS, CH)

    part, deg = _sc_conv(edges2d, w2d, feat)

    p0 = part[0].reshape(B, S, D_IN)
    p1 = part[1].reshape(B, S, D_IN)
    d0p = jnp.pad(deg[0, :N].reshape(B, S), ((0, 0), (0, 128 - S)))
    d1p = jnp.pad(deg[1, :N].reshape(B, S), ((0, 0), (0, 128 - S)))
    fa = feat.reshape(B, S, D_IN)[:, 0, :]
    b1p = jnp.broadcast_to(b1.reshape(1, D_OUT), (8, D_OUT))
    b2p = jnp.broadcast_to(b2.reshape(1, D_OUT), (8, D_OUT))
    ap = jnp.broadcast_to(prelu_a.reshape(1, 1), (8, 128))

    pool, rec, ao1, ao2 = _tc_readout(p0, p1, d0p, d1p, fa, W1, W2, b1p, b2p, ap)
    return pool, rec, ao1, ao2


# speedup vs baseline: 8.3425x; 1.3142x over previous
import functools

import jax
import jax.numpy as jnp
from jax import lax
from jax.experimental import pallas as pl
from jax.experimental.pallas import tpu as pltpu
from jax.experimental.pallas import tpu_sc as plsc

N = 10000
E = 320000
D_IN = 128
D_OUT = 256
S = 10
B = N // S

NC = 2
NS = 16
NW = NC * NS

CH = 100
ROWS = E // CH
RPW = ROWS // NW
RPT = ROWS // NS
CHV = 128
NPAD = 10240
HPT = NPAD // NS
ACC_RPT = N // NS

def _rsqrt16(x):
    i = plsc.bitcast(x, jnp.int32)
    i = jnp.int32(0x5F3759DF) - lax.shift_right_logical(i, 1)
    y = plsc.bitcast(i, jnp.float32)
    for _ in range(3):
        y = y * (1.5 - 0.5 * x * y * y)
    return y


GP = 20

_OFFS = (0, 16, 32, 48, 64, 80, CH - 16)


def _sc_conv_body(edges_hbm, w_hbm, feat_hbm,
                  part_hbm, deg_hbm,
                  ig_v, w_v, ones_v, nb, rows0, rows1, rows2, wp_v, rv_c,
                  hist_sh, dhist_sh, acc_sh,
                  sem0, sem1, sem2, ssem0, ssem1, ssem2):
    c = lax.axis_index("c")
    s = lax.axis_index("s")
    wid = s * NC + c

    zeros16 = jnp.zeros((16,), jnp.float32)
    ones16 = jnp.ones((16,), jnp.float32)
    for k in range(CHV // 16):
        ones_v[pl.ds(k * 16, 16)] = ones16

    def _z_nb(i, _):
        nb[pl.ds(i * 16, 16)] = zeros16
        return ()
    lax.fori_loop(0, HPT // 16, _z_nb, ())

    def _z_rows(i, _):
        for q in range(D_IN // 16):
            rows0[i, pl.ds(q * 16, 16)] = zeros16
        return ()
    lax.fori_loop(0, CH, _z_rows, ())

    pltpu.sync_copy(nb, hist_sh.at[pl.ds(s * HPT, HPT)])
    pltpu.sync_copy(nb, dhist_sh.at[pl.ds(s * HPT, HPT)])
    for t in range(ACC_RPT // CH):
        pltpu.sync_copy(rows0, acc_sh.at[pl.ds(s * ACC_RPT + t * CH, CH)])
    pltpu.sync_copy(rows0.at[pl.ds(0, ACC_RPT % CH)],
                    acc_sh.at[pl.ds(s * ACC_RPT + ACC_RPT - ACC_RPT % CH,
                                    ACC_RPT % CH)])
    plsc.subcore_barrier()

    def _hist(plane, base, n_groups, hist):
        def _outer(g, _):
            pltpu.sync_copy(edges_hbm.at[plane, pl.ds(base + g * GP, GP)],
                            ig_v.at[pl.ds(0, GP)])
            descs = [pltpu.async_copy(ones_v.at[pl.ds(0, CH)],
                                      hist.at[ig_v.at[j]], sem0, add=True)
                     for j in range(GP)]
            for d in descs:
                d.wait()
            return ()
        lax.fori_loop(0, n_groups, _outer, ())

    _hist(0, s * RPT, RPT // GP, hist_sh)
    _hist(1, c * (ROWS // NC) + s * RPW, RPW // GP, dhist_sh)
    plsc.subcore_barrier()

    pltpu.sync_copy(dhist_sh.at[pl.ds(s * HPT, HPT)],
                    deg_hbm.at[c, pl.ds(s * HPT, HPT)])
    pltpu.sync_copy(hist_sh.at[pl.ds(s * HPT, HPT)], nb)

    def _newton(i, _):
        x = nb[pl.ds(i * 16, 16)]
        x = jnp.maximum(x, 1.0)
        nb[pl.ds(i * 16, 16)] = _rsqrt16(x)
        return ()
    lax.fori_loop(0, HPT // 16, _newton, ())
    pltpu.sync_copy(nb, hist_sh.at[pl.ds(s * HPT, HPT)])
    plsc.subcore_barrier()

    bufs = (rows0, rows1, rows2)
    gsems = (sem0, sem1, sem2)
    ssems = (ssem0, ssem1, ssem2)

    def _group(g, _):
        base = wid * RPW + g * GP
        pltpu.sync_copy(edges_hbm.at[0, pl.ds(base, GP)], ig_v.at[pl.ds(0, GP)])
        pltpu.sync_copy(edges_hbm.at[1, pl.ds(base, GP)], ig_v.at[pl.ds(GP, GP)])
        pltpu.sync_copy(w_hbm.at[pl.ds(base, GP)], w_v)

        gd = {0: pltpu.async_copy(feat_hbm.at[ig_v.at[0]], bufs[0], gsems[0])}
        sd = {}
        for j2 in range(GP):
            cur = bufs[j2 % 3]
            if j2 - 2 in sd:
                sd.pop(j2 - 2).wait()
            if j2 + 1 < GP:
                gd[j2 + 1] = pltpu.async_copy(
                    feat_hbm.at[ig_v.at[j2 + 1]], bufs[(j2 + 1) % 3],
                    gsems[(j2 + 1) % 3])

            pltpu.sync_copy(hist_sh.at[ig_v.at[j2]], rv_c.at[pl.ds(0, CH)])
            for off in _OFFS:
                sv = ig_v[j2, pl.ds(off, 16)]
                gg = rv_c[pl.ds(off, 16)]
                w16 = w_v[j2, pl.ds(off, 16)]
                keep = lax.rem(sv, jnp.int32(S)) != 0
                wp_v[pl.ds(off, 16)] = jnp.where(keep, w16 * gg, 0.0)

            gd.pop(j2).wait()

            @plsc.parallel_loop(0, CH, step=1, unroll=4)
            def _scale(e, cur=cur):
                g16 = plsc.load_gather(wp_v, [jnp.full((16,), e, jnp.int32)])
                for q in range(D_IN // 16):
                    cur[e, pl.ds(q * 16, 16)] = cur[e, pl.ds(q * 16, 16)] * g16

            sd[j2] = pltpu.async_copy(cur, acc_sh.at[ig_v.at[GP + j2]],
                                      ssems[j2 % 3], add=True)
        sd.pop(GP - 2).wait()
        sd.pop(GP - 1).wait()
        return ()
    lax.fori_loop(0, RPW // GP, _group, ())
    plsc.subcore_barrier()

    for t in range(ACC_RPT // CH):
        blk = pl.ds(s * ACC_RPT + t * CH, CH)
        pltpu.sync_copy(acc_sh.at[blk], rows0)
        pltpu.sync_copy(rows0, part_hbm.at[c, blk])
    rem = ACC_RPT % CH
    blk = pl.ds(s * ACC_RPT + ACC_RPT - rem, rem)
    pltpu.sync_copy(acc_sh.at[blk], rows0.at[pl.ds(0, rem)])
    pltpu.sync_copy(rows0.at[pl.ds(0, rem)], part_hbm.at[c, blk])


@functools.partial(jax.jit, static_argnames=())
def _sc_conv(edges2d, w2d, feat):
    mesh = plsc.VectorSubcoreMesh(core_axis_name="c", subcore_axis_name="s",
                                  num_cores=NC, num_subcores=NS)
    fn = pl.kernel(
        _sc_conv_body,
        out_type=(
            jax.ShapeDtypeStruct((NC, N, D_IN), jnp.float32),
            jax.ShapeDtypeStruct((NC, NPAD), jnp.float32),
        ),
        mesh=mesh,
        scratch_types=[
            pltpu.VMEM((2 * GP, CH), jnp.int32),
            pltpu.VMEM((GP, CH), jnp.float32),
            pltpu.VMEM((CHV,), jnp.float32),
            pltpu.VMEM((HPT,), jnp.float32),
            pltpu.VMEM((CH, D_IN), jnp.float32),
            pltpu.VMEM((CH, D_IN), jnp.float32),
            pltpu.VMEM((CH, D_IN), jnp.float32),
            pltpu.VMEM((CHV,), jnp.float32),
            pltpu.VMEM((CHV,), jnp.float32),
            pltpu.VMEM_SHARED((NPAD,), jnp.float32),
            pltpu.VMEM_SHARED((NPAD,), jnp.float32),
            pltpu.VMEM_SHARED((N, D_IN), jnp.float32),
            pltpu.SemaphoreType.DMA,
            pltpu.SemaphoreType.DMA,
            pltpu.SemaphoreType.DMA,
            pltpu.SemaphoreType.DMA,
            pltpu.SemaphoreType.DMA,
            pltpu.SemaphoreType.DMA,
        ],
        compiler_params=pltpu.CompilerParams(use_tc_tiling_on_sc=False,
                                             needs_layout_passes=False),
    )
    return fn(edges2d, w2d, feat)


G = 200
GRID = B // G


def _tc_readout_body(p0_ref, p1_ref, d0_ref, d1_ref, fa_ref,
                     w1_ref, w2_ref, b1_ref, b2_ref, a_ref,
                     pool_ref, rec_ref, ao1_ref, ao2_ref):
    a = a_ref[0, 0]
    w1 = w1_ref[...]
    w2 = w2_ref[...]
    b1 = b1_ref[0:1, :]
    b2 = b2_ref[0:1, :]

    def prelu(x):
        return jnp.where(x >= 0, x, a * x)

    def l2n(x):
        n = jnp.sqrt(jnp.sum(x * x, axis=1, keepdims=True))
        return x / jnp.maximum(n, 1e-12)

    d = d0_ref[:, :S] + d1_ref[:, :S]
    rin = lax.rsqrt(jnp.maximum(d, 1.0))

    acc = jnp.zeros((G, D_OUT), jnp.float32)
    rec = None
    for si in range(S):
        h = (p0_ref[:, si, :] + p1_ref[:, si, :]) * rin[:, si:si + 1]
        if si == 0:
            rec = prelu(jnp.dot(h, w2, preferred_element_type=jnp.float32) + b2)
        else:
            acc = acc + prelu(jnp.dot(h, w1, preferred_element_type=jnp.float32) + b1)
    pool = acc * jnp.float32(1.0 / (S - 1))

    fa = fa_ref[...]
    ao1 = prelu(jnp.dot(fa, w1, preferred_element_type=jnp.float32) + b1)
    ao2 = prelu(jnp.dot(fa, w2, preferred_element_type=jnp.float32) + b2)

    pool_ref[...] = l2n(pool)
    rec_ref[...] = l2n(rec)
    ao1_ref[...] = l2n(ao1)
    ao2_ref[...] = l2n(ao2)


def _tc_readout(p0, p1, d0p, d1p, fa, W1, W2, b1p, b2p, ap):
    out_sds = jax.ShapeDtypeStruct((B, D_OUT), jnp.float32)
    return pl.pallas_call(
        _tc_readout_body,
        grid=(GRID,),
        in_specs=[
            pl.BlockSpec((G, S, D_IN), lambda i: (i, 0, 0)),
            pl.BlockSpec((G, S, D_IN), lambda i: (i, 0, 0)),
            pl.BlockSpec((G, 128), lambda i: (i, 0)),
            pl.BlockSpec((G, 128), lambda i: (i, 0)),
            pl.BlockSpec((G, D_IN), lambda i: (i, 0)),
            pl.BlockSpec((D_IN, D_OUT), lambda i: (0, 0)),
            pl.BlockSpec((D_IN, D_OUT), lambda i: (0, 0)),
            pl.BlockSpec((8, D_OUT), lambda i: (0, 0)),
            pl.BlockSpec((8, D_OUT), lambda i: (0, 0)),
            pl.BlockSpec((8, 128), lambda i: (0, 0)),
        ],
        out_specs=[pl.BlockSpec((G, D_OUT), lambda i: (i, 0))] * 4,
        out_shape=[out_sds] * 4,
    )(p0, p1, d0p, d1p, fa, W1, W2, b1p, b2p, ap)


def kernel(feat, edge_index, edge_weight, W1, W2, b1, b2, prelu_a, subgraph_size):
    del subgraph_size

    edges2d = edge_index.reshape(2, ROWS, CH)
    w2d = edge_weight.reshape(ROWS, CH)

    part, deg = _sc_conv(edges2d, w2d, feat)

    p0 = part[0].reshape(B, S, D_IN)
    p1 = part[1].reshape(B, S, D_IN)
    d0p = jnp.pad(deg[0, :N].reshape(B, S), ((0, 0), (0, 128 - S)))
    d1p = jnp.pad(deg[1, :N].reshape(B, S), ((0, 0), (0, 128 - S)))
    fa = feat.reshape(B, S, D_IN)[:, 0, :]
    b1p = jnp.broadcast_to(b1.reshape(1, D_OUT), (8, D_OUT))
    b2p = jnp.broadcast_to(b2.reshape(1, D_OUT), (8, D_OUT))
    ap = jnp.broadcast_to(prelu_a.reshape(1, 1), (8, 128))

    pool, rec, ao1, ao2 = _tc_readout(p0, p1, d0p, d1p, fa, W1, W2, b1p, b2p, ap)
    return pool, rec, ao1, ao2
